# baseline (speedup 1.0000x reference)
import functools

import jax
import jax.numpy as jnp
from jax.experimental import pallas as pl
from jax.experimental.pallas import tpu as pltpu


def _enc_kernel(terms_sm, tvec_ref, cnts_ref, wtab_ref, etab_ref, out_ref,
                erows, wrows, *, L, VH, D):
    j = pl.program_id(0)
    b = pl.program_id(1)
    vbase = j * VH
    tb = b * L

    for l in range(L):
        idx = terms_sm[tb + l]
        il = jnp.clip(idx - vbase, 0, VH - 1)
        erows[l] = etab_ref[il]
        wrows[l] = wtab_ref[idx // 128]

    E = erows[...]
    W = wrows[...]

    tvec = tvec_ref[b]
    lane = jax.lax.broadcasted_iota(jnp.int32, (L, 1, 128), 2)
    lp = jnp.reshape(tvec % 128, (L, 1, 1))
    w_tok = jnp.sum(jnp.where(lane == lp, W, 0.0), axis=2)

    logits = w_tok + jnp.log(cnts_ref[b])
    m = jnp.max(logits)
    e = jnp.exp(logits - m)
    s = jnp.sum(e)

    keep = (tvec >= vbase) & (tvec < vbase + VH)
    ew = jnp.where(keep, e, 0.0) * (1.0 / s)
    acc = jnp.sum(jnp.reshape(ew, (L, 1, 1)) * E, axis=0)
    out_ref[0, b] = acc


def kernel(terms, cnts, weights_table, embed_table):
    B, L = terms.shape
    V, D = embed_table.shape
    VH = V // 2
    NW = V // 128

    etab = embed_table.astype(jnp.float32).reshape(V, 1, D)
    wtab = weights_table.astype(jnp.float32).reshape(NW, 1, 128)
    tflat = terms.astype(jnp.int32).reshape(-1)
    t3 = terms.astype(jnp.int32).reshape(B, L, 1)
    c3 = cnts.astype(jnp.float32).reshape(B, L, 1)

    kernel_fn = functools.partial(_enc_kernel, L=L, VH=VH, D=D)

    out = pl.pallas_call(
        kernel_fn,
        out_shape=jax.ShapeDtypeStruct((2, B, 1, D), jnp.float32),
        grid_spec=pltpu.PrefetchScalarGridSpec(
            num_scalar_prefetch=1,
            grid=(2, B),
            in_specs=[
                pl.BlockSpec((B, L, 1), lambda j, b, t: (0, 0, 0)),
                pl.BlockSpec((B, L, 1), lambda j, b, t: (0, 0, 0)),
                pl.BlockSpec((NW, 1, 128), lambda j, b, t: (0, 0, 0)),
                pl.BlockSpec((VH, 1, D), lambda j, b, t: (j, 0, 0)),
            ],
            out_specs=pl.BlockSpec((1, B, 1, D), lambda j, b, t: (j, 0, 0, 0)),
            scratch_shapes=[
                pltpu.VMEM((L, 1, D), jnp.float32),
                pltpu.VMEM((L, 1, 128), jnp.float32),
            ],
        ),
        compiler_params=pltpu.CompilerParams(
            dimension_semantics=("parallel", "arbitrary"),
            vmem_limit_bytes=32 * 1024 * 1024,
        ),
    )(tflat, t3, c3, wtab, etab)

    return out[0, :, 0, :] + out[1, :, 0, :]


# speedup vs baseline: 2.6818x; 2.6818x over previous
import functools

import jax
import jax.numpy as jnp
from jax.experimental import pallas as pl
from jax.experimental.pallas import tpu as pltpu


def _enc_kernel(terms_sm, tvec_ref, cnts_ref, wtab_ref, etab_ref, out_ref,
                erows, wrows, *, B, L, VH, D):
    j = pl.program_id(0)
    vbase = j * VH

    for t in range(B * L):
        idx = terms_sm[t]
        il = jnp.clip(idx - vbase, 0, VH - 1)
        erows[pl.ds(t, 1), :] = etab_ref[pl.ds(il, 1), :]
        wrows[pl.ds(t, 1), :] = wtab_ref[pl.ds(idx // 128, 1), :]

    G = erows[...].reshape(B, L, D)
    W = wrows[...].reshape(B, L, 128)
    tvec = tvec_ref[...]

    lane = jax.lax.broadcasted_iota(jnp.int32, (B, L, 128), 2)
    w_tok = jnp.sum(jnp.where(lane == tvec % 128, W, 0.0),
                    axis=2, keepdims=True)

    logits = w_tok + jnp.log(cnts_ref[...])
    m = jnp.max(logits, axis=1, keepdims=True)
    e = jnp.exp(logits - m)
    s = jnp.sum(e, axis=1, keepdims=True)

    keep = (tvec >= vbase) & (tvec < vbase + VH)
    ew = jnp.where(keep, e, 0.0) / s
    out_ref[0] = jnp.sum(ew * G, axis=1)


def kernel(terms, cnts, weights_table, embed_table):
    B, L = terms.shape
    V, D = embed_table.shape
    VH = V // 2
    NW = V // 128

    wtab = weights_table.astype(jnp.float32).reshape(NW, 128)
    tflat = terms.astype(jnp.int32).reshape(-1)
    t3 = terms.astype(jnp.int32).reshape(B, L, 1)
    c3 = cnts.astype(jnp.float32).reshape(B, L, 1)

    kernel_fn = functools.partial(_enc_kernel, B=B, L=L, VH=VH, D=D)

    out = pl.pallas_call(
        kernel_fn,
        out_shape=jax.ShapeDtypeStruct((2, B, D), jnp.float32),
        grid_spec=pltpu.PrefetchScalarGridSpec(
            num_scalar_prefetch=1,
            grid=(2,),
            in_specs=[
                pl.BlockSpec((B, L, 1), lambda j, t: (0, 0, 0)),
                pl.BlockSpec((B, L, 1), lambda j, t: (0, 0, 0)),
                pl.BlockSpec((NW, 128), lambda j, t: (0, 0)),
                pl.BlockSpec((VH, D), lambda j, t: (j, 0)),
            ],
            out_specs=pl.BlockSpec((1, B, D), lambda j, t: (j, 0, 0)),
            scratch_shapes=[
                pltpu.VMEM((B * L, D), jnp.float32),
                pltpu.VMEM((B * L, 128), jnp.float32),
            ],
        ),
        compiler_params=pltpu.CompilerParams(
            dimension_semantics=("parallel",),
            vmem_limit_bytes=32 * 1024 * 1024,
        ),
    )(tflat, t3, c3, wtab, embed_table.astype(jnp.float32))

    return out[0] + out[1]


# speedup vs baseline: 2.9642x; 1.1053x over previous
import functools

import jax
import jax.numpy as jnp
from jax.experimental import pallas as pl
from jax.experimental.pallas import tpu as pltpu


def _enc_kernel(terms_sm, tvec_ref, cnts_ref, wtab_ref, etab_hbm, out_ref,
                erows, wrows, sem, *, BH, L, D):
    j = pl.program_id(0)
    M = BH * L
    base = j * M

    for t in range(M):
        idx = terms_sm[base + t]
        pltpu.make_async_copy(
            etab_hbm.at[pl.ds(idx, 1), :],
            erows.at[pl.ds(t, 1), :],
            sem,
        ).start()

    for t in range(M):
        idx = terms_sm[base + t]
        wrows[pl.ds(t, 1), :] = wtab_ref[pl.ds(idx // 128, 1), :]

    pltpu.make_async_copy(
        etab_hbm.at[pl.ds(0, M), :], erows.at[pl.ds(0, M), :], sem,
    ).wait()

    W = wrows[...].reshape(BH, L, 128)
    tvec = tvec_ref[0].reshape(BH, L, 1)

    lane = jax.lax.broadcasted_iota(jnp.int32, (BH, L, 128), 2)
    w_tok = jnp.sum(jnp.where(lane == tvec % 128, W, 0.0),
                    axis=2, keepdims=True)

    logits = w_tok + jnp.log(cnts_ref[0].reshape(BH, L, 1))
    m = jnp.max(logits, axis=1, keepdims=True)
    e = jnp.exp(logits - m)
    s = jnp.sum(e, axis=1, keepdims=True)
    p = e / s

    G = erows[...].reshape(BH, L, D)
    out_ref[0] = jnp.sum(p * G, axis=1)


def kernel(terms, cnts, weights_table, embed_table):
    B, L = terms.shape
    V, D = embed_table.shape
    BH = B // 2
    NW = V // 128

    wtab = weights_table.astype(jnp.float32).reshape(NW, 128)
    tflat = terms.astype(jnp.int32).reshape(-1)
    t3 = terms.astype(jnp.int32).reshape(2, BH * L, 1)
    c3 = cnts.astype(jnp.float32).reshape(2, BH * L, 1)

    kernel_fn = functools.partial(_enc_kernel, BH=BH, L=L, D=D)

    out = pl.pallas_call(
        kernel_fn,
        out_shape=jax.ShapeDtypeStruct((2, BH, D), jnp.float32),
        grid_spec=pltpu.PrefetchScalarGridSpec(
            num_scalar_prefetch=1,
            grid=(2,),
            in_specs=[
                pl.BlockSpec((1, BH * L, 1), lambda j, t: (j, 0, 0)),
                pl.BlockSpec((1, BH * L, 1), lambda j, t: (j, 0, 0)),
                pl.BlockSpec((NW, 128), lambda j, t: (0, 0)),
                pl.BlockSpec(memory_space=pl.ANY),
            ],
            out_specs=pl.BlockSpec((1, BH, D), lambda j, t: (j, 0, 0)),
            scratch_shapes=[
                pltpu.VMEM((BH * L, D), jnp.float32),
                pltpu.VMEM((BH * L, 128), jnp.float32),
                pltpu.SemaphoreType.DMA,
            ],
        ),
        compiler_params=pltpu.CompilerParams(
            dimension_semantics=("parallel",),
            vmem_limit_bytes=32 * 1024 * 1024,
        ),
    )(tflat, t3, c3, wtab, embed_table.astype(jnp.float32))

    return out.reshape(B, D)
